# baseline (speedup 1.0000x reference)
import jax
import jax.numpy as jnp
import numpy as np
from jax import lax
from jax.experimental import pallas as pl
from jax.experimental.pallas import tpu as pltpu
from jax.experimental.pallas import tpu_sc as plsc

NC = 2
NS = 16
NW = NC * NS
CH = 128
SB = 8


def _sc_mesh():
    return plsc.VectorSubcoreMesh(core_axis_name="c", subcore_axis_name="s")


def _degree_call(np_, tpc):
    def body(idx2, degp, idx_v, dga, dgb):
        c = lax.axis_index("c")
        s = lax.axis_index("s")
        wid = c * NS + s
        pltpu.sync_copy(idx2.at[wid], idx_v)

        zv = jnp.zeros((16,), jnp.float32)

        def zstep(i, carry):
            dga[pl.ds(i * 16, 16)] = zv
            dgb[pl.ds(i * 16, 16)] = zv
            return carry

        lax.fori_loop(0, np_ // 16, zstep, 0)

        ones = jnp.ones((16,), jnp.float32)

        def estep(g, carry):
            for k in range(CH // 16):
                va = idx_v[2 * g, pl.ds(k * 16, 16)]
                plsc.addupdate_scatter(dga, [va], ones)
            for k in range(CH // 16):
                vb = idx_v[2 * g + 1, pl.ds(k * 16, 16)]
                plsc.addupdate_scatter(dgb, [vb], ones)
            return carry

        lax.fori_loop(0, tpc, estep, 0)
        pltpu.sync_copy(dga, degp.at[c, s, 0])
        pltpu.sync_copy(dgb, degp.at[c, s, 1])

    return pl.kernel(
        body,
        out_type=jax.ShapeDtypeStruct((NC, NS, 2, np_), jnp.float32),
        mesh=_sc_mesh(),
        compiler_params=pltpu.CompilerParams(needs_layout_passes=False),
        scratch_types=[
            pltpu.VMEM((2 * tpc, CH), jnp.int32),
            pltpu.VMEM((np_,), jnp.float32),
            pltpu.VMEM((np_,), jnp.float32),
        ],
    )


def _unpack_rows(rows16, rows32, d, lo, hi):
    def row(r, carry):
        for k in range(d // 32):
            v = rows16[r, pl.ds(k * 16, 16)]
            vb = plsc.bitcast(v, jnp.bfloat16)
            a, b = plsc.unpack(vb, format=plsc.PackFormat.INTERLEAVED)
            rows32[r, pl.ds(k * 32, 16)] = a
            rows32[r, pl.ds(k * 32 + 16, 16)] = b
        return carry

    lax.fori_loop(lo, hi, row, 0)


def _edge_call(np_, nsb0, nsb1, d):
    rpt = np_ // NS
    HF = CH // 2

    def body(src4, dst4, h, z128, accp,
             sbufa, sbufb, dbufa, dbufb, rows0, rows1, rows32, acc,
             sa, sb_, s0, s1, ss0, ss1):
        c = lax.axis_index("c")
        s = lax.axis_index("s")
        wid = c * NS + s
        pltpu.sync_copy(src4.at[wid, 0], sbufa)
        pltpu.sync_copy(dst4.at[wid, 0], dbufa)
        pltpu.async_copy(src4.at[wid, 1], sbufb, sb_)
        pltpu.async_copy(dst4.at[wid, 1], dbufb, sb_)
        r0 = s * rpt
        pltpu.sync_copy(z128, acc.at[pl.ds(r0, rpt)])
        plsc.subcore_barrier()

        half = jnp.where(c == 0, nsb0 // 2, nsb1 // 2)

        def half_wait(sem):
            pltpu.make_async_copy(
                rows32.at[pl.ds(0, HF)], accp.at[0, pl.ds(0, HF)], sem).wait()

        def process(sbuf, dbuf, prev):
            pltpu.async_copy(h.at[sbuf.at[0]], rows0, s0)
            for k in range(SB):
                rw, sw = (rows0, s0) if k % 2 == 0 else (rows1, s1)
                pltpu.make_async_copy(h.at[sbuf.at[k]], rw, sw).wait()
                if k + 1 < SB:
                    nrw, nsw = (rows1, s1) if k % 2 == 0 else (rows0, s0)
                    pltpu.async_copy(h.at[sbuf.at[k + 1]], nrw, nsw)
                for hh, ssem in ((0, ss0), (1, ss1)):
                    if k > 0 or prev is True:
                        half_wait(ssem)
                    elif prev is not False:
                        @pl.when(prev)
                        def _():
                            half_wait(ssem)
                    _unpack_rows(rw, rows32, d, hh * HF, (hh + 1) * HF)
                    pltpu.async_copy(rows32.at[pl.ds(hh * HF, HF)],
                                     acc.at[dbuf.at[2 * k + hh]], ssem,
                                     add=True)

        def step(g, carry):
            @pl.when(g > 0)
            def _():
                pltpu.make_async_copy(src4.at[wid, 0], sbufa, sa).wait()
                pltpu.make_async_copy(dst4.at[wid, 0], dbufa, sa).wait()

            process(sbufa, dbufa, g > 0)

            @pl.when(g + 1 < half)
            def _():
                pltpu.async_copy(src4.at[wid, 2 * g + 2], sbufa, sa)
                pltpu.async_copy(dst4.at[wid, 2 * g + 2], dbufa, sa)

            pltpu.make_async_copy(src4.at[wid, 1], sbufb, sb_).wait()
            pltpu.make_async_copy(dst4.at[wid, 1], dbufb, sb_).wait()
            process(sbufb, dbufb, True)

            @pl.when(g + 1 < half)
            def _():
                pltpu.async_copy(src4.at[wid, 2 * g + 3], sbufb, sb_)
                pltpu.async_copy(dst4.at[wid, 2 * g + 3], dbufb, sb_)

            return carry

        lax.fori_loop(0, half, step, 0)
        half_wait(ss0)
        half_wait(ss1)
        plsc.subcore_barrier()
        pltpu.sync_copy(acc.at[pl.ds(r0, rpt)], accp.at[c, pl.ds(r0, rpt)])

    return pl.kernel(
        body,
        out_type=jax.ShapeDtypeStruct((NC, np_, d), jnp.float32),
        mesh=_sc_mesh(),
        compiler_params=pltpu.CompilerParams(needs_layout_passes=False,
                                             use_tc_tiling_on_sc=False),
        scratch_types=[
            pltpu.VMEM((SB, CH), jnp.int32),
            pltpu.VMEM((SB, CH), jnp.int32),
            pltpu.VMEM((2 * SB, CH // 2), jnp.int32),
            pltpu.VMEM((2 * SB, CH // 2), jnp.int32),
            pltpu.VMEM((CH, d // 2), jnp.int32),
            pltpu.VMEM((CH, d // 2), jnp.int32),
            pltpu.VMEM((CH, d), jnp.float32),
            pltpu.VMEM_SHARED((np_, d), jnp.float32),
            pltpu.SemaphoreType.DMA,
            pltpu.SemaphoreType.DMA,
            pltpu.SemaphoreType.DMA,
            pltpu.SemaphoreType.DMA,
            pltpu.SemaphoreType.DMA,
            pltpu.SemaphoreType.DMA,
        ],
    )


def _norms(dvec):
    d0 = dvec[:, None]
    return jnp.where(d0 > 0, lax.rsqrt(jnp.maximum(d0, 1.0)), 0.0)


def _tc_pre(np_, d, blk):
    grid = np_ // blk

    def body(degp_ref, x_ref, w_ref, ns_ref, nd_ref, h_ref):
        dp = degp_ref[...]
        ns = _norms(dp[:, :, 0, :].sum((0, 1)))
        nd = _norms(dp[:, :, 1, :].sum((0, 1)))
        ns_ref[...] = ns
        nd_ref[...] = nd
        h_ref[...] = jnp.dot(x_ref[...] * ns, w_ref[...],
                             preferred_element_type=jnp.float32
                             ).astype(jnp.bfloat16)

    return pl.pallas_call(
        body,
        grid=(grid,),
        in_specs=[
            pl.BlockSpec((NC, NS, 2, blk), lambda i: (0, 0, 0, i)),
            pl.BlockSpec((blk, d), lambda i: (i, 0)),
            pl.BlockSpec((d, d), lambda i: (0, 0)),
        ],
        out_specs=[
            pl.BlockSpec((blk, 1), lambda i: (i, 0)),
            pl.BlockSpec((blk, 1), lambda i: (i, 0)),
            pl.BlockSpec((blk, d), lambda i: (i, 0)),
        ],
        out_shape=[
            jax.ShapeDtypeStruct((np_, 1), jnp.float32),
            jax.ShapeDtypeStruct((np_, 1), jnp.float32),
            jax.ShapeDtypeStruct((np_, d), jnp.bfloat16),
        ],
    )


def _tc_mid(np_, d, blk):
    grid = np_ // blk

    def body(accp_ref, ns_ref, nd_ref, b_ref, w_ref, h_ref):
        ap = accp_ref[...]
        z = jnp.maximum((ap[0] + ap[1]) * nd_ref[...] + b_ref[...], 0.0)
        h_ref[...] = jnp.dot(z * ns_ref[...], w_ref[...],
                             preferred_element_type=jnp.float32
                             ).astype(jnp.bfloat16)

    return pl.pallas_call(
        body,
        grid=(grid,),
        in_specs=[
            pl.BlockSpec((NC, blk, d), lambda i: (0, i, 0)),
            pl.BlockSpec((blk, 1), lambda i: (i, 0)),
            pl.BlockSpec((blk, 1), lambda i: (i, 0)),
            pl.BlockSpec((1, d), lambda i: (0, 0)),
            pl.BlockSpec((d, d), lambda i: (0, 0)),
        ],
        out_specs=pl.BlockSpec((blk, d), lambda i: (i, 0)),
        out_shape=jax.ShapeDtypeStruct((np_, d), jnp.bfloat16),
    )


def _tc_post(n, d, blk):
    grid = n // blk

    def body(accp_ref, nd_ref, b_ref, out_ref):
        ap = accp_ref[...]
        out_ref[...] = jnp.maximum((ap[0] + ap[1]) * nd_ref[...] + b_ref[...], 0.0)

    return pl.pallas_call(
        body,
        grid=(grid,),
        in_specs=[
            pl.BlockSpec((NC, blk, d), lambda i: (0, i, 0)),
            pl.BlockSpec((blk, 1), lambda i: (i, 0)),
            pl.BlockSpec((1, d), lambda i: (0, 0)),
        ],
        out_specs=pl.BlockSpec((blk, d), lambda i: (i, 0)),
        out_shape=jax.ShapeDtypeStruct((n, d), jnp.float32),
    )


def kernel(x, edge_index, W1, b1, W2, b2):
    n, d = x.shape
    e = edge_index.shape[1]

    np_ = ((n + 1 + 2047) // 2048) * 2048
    gran = NW * CH * SB * 2
    ep = ((e + gran - 1) // gran) * gran
    tpc = ep // (NW * CH)
    nsb = tpc // SB
    nsb0 = 2 * max(2, int(round(nsb * 2 * 0.4 / 2)))
    nsb1 = 2 * nsb - nsb0
    rpt = np_ // NS

    pad = jnp.full((ep - e,), n, dtype=jnp.int32)
    src3 = jnp.concatenate([edge_index[0], pad]).reshape(NW, tpc, CH)
    dst3 = jnp.concatenate([edge_index[1], pad]).reshape(NW, tpc, CH)
    idx2 = jnp.stack([src3, dst3], axis=2).reshape(NW, 2 * tpc, CH)

    spe = SB * CH
    cut = NS * nsb0 * spe
    srcp = jnp.concatenate([edge_index[0], pad])
    dstp = jnp.concatenate([edge_index[1], pad])

    def uneven(v):
        a = v[:cut].reshape(NS, nsb0, spe)
        a = jnp.pad(a, ((0, 0), (0, nsb1 - nsb0), (0, 0)), constant_values=n)
        b = v[cut:].reshape(NS, nsb1, spe)
        return jnp.concatenate([a, b], axis=0)

    src4 = uneven(srcp).reshape(NW, nsb1, SB, CH)
    dst4 = uneven(dstp).reshape(NW, nsb1, 2 * SB, CH // 2)

    z128 = jnp.zeros((rpt, d), dtype=jnp.float32)
    xp = jnp.pad(x, ((0, np_ - n), (0, 0)))
    b1r = b1.reshape(1, d)
    b2r = b2.reshape(1, d)

    perm = []
    for g in range(d // 32):
        perm += [32 * g + 2 * i for i in range(16)]
        perm += [32 * g + 2 * i + 1 for i in range(16)]
    q = np.argsort(np.asarray(perm))
    W1q = W1[:, q]
    W2q = W2[:, q]

    def pack_i32(hb):
        return lax.bitcast_convert_type(hb.reshape(np_, d // 2, 2), jnp.int32)

    degp = _degree_call(np_, tpc)(idx2)
    ns, nd, h1 = _tc_pre(np_, d, 1024)(degp, xp, W1q)
    acc1 = _edge_call(np_, nsb0, nsb1, d)(src4, dst4, pack_i32(h1), z128)
    h2 = _tc_mid(np_, d, 1024)(acc1, ns, nd, b1r, W2q)
    acc2 = _edge_call(np_, nsb0, nsb1, d)(src4, dst4, pack_i32(h2), z128)
    out = _tc_post(n, d, 1000)(acc2, nd, b2r)
    return out


# speedup vs baseline: 1.1300x; 1.1300x over previous
import jax
import jax.numpy as jnp
import numpy as np
from jax import lax
from jax.experimental import pallas as pl
from jax.experimental.pallas import tpu as pltpu
from jax.experimental.pallas import tpu_sc as plsc

NC = 2
NS = 16
NW = NC * NS
CH = 128
SB = 8


def _sc_mesh():
    return plsc.VectorSubcoreMesh(core_axis_name="c", subcore_axis_name="s")


def _degree_call(np_, tpc):
    def body(idx2, degp, idx_v, dga, dgb):
        c = lax.axis_index("c")
        s = lax.axis_index("s")
        wid = c * NS + s
        pltpu.sync_copy(idx2.at[wid], idx_v)

        zv = jnp.zeros((16,), jnp.float32)

        def zstep(i, carry):
            dga[pl.ds(i * 16, 16)] = zv
            dgb[pl.ds(i * 16, 16)] = zv
            return carry

        lax.fori_loop(0, np_ // 16, zstep, 0)

        ones = jnp.ones((16,), jnp.float32)

        def estep(g, carry):
            for k in range(CH // 16):
                va = idx_v[2 * g, pl.ds(k * 16, 16)]
                plsc.addupdate_scatter(dga, [va], ones)
            for k in range(CH // 16):
                vb = idx_v[2 * g + 1, pl.ds(k * 16, 16)]
                plsc.addupdate_scatter(dgb, [vb], ones)
            return carry

        lax.fori_loop(0, tpc, estep, 0)
        pltpu.sync_copy(dga, degp.at[c, s, 0])
        pltpu.sync_copy(dgb, degp.at[c, s, 1])

    return pl.kernel(
        body,
        out_type=jax.ShapeDtypeStruct((NC, NS, 2, np_), jnp.float32),
        mesh=_sc_mesh(),
        compiler_params=pltpu.CompilerParams(needs_layout_passes=False),
        scratch_types=[
            pltpu.VMEM((2 * tpc, CH), jnp.int32),
            pltpu.VMEM((np_,), jnp.float32),
            pltpu.VMEM((np_,), jnp.float32),
        ],
    )


def _unpack_rows(rows16, rows32, d, lo, hi):
    def row(r, carry):
        for k in range(d // 32):
            v = rows16[r, pl.ds(k * 16, 16)]
            vb = plsc.bitcast(v, jnp.bfloat16)
            a, b = plsc.unpack(vb, format=plsc.PackFormat.INTERLEAVED)
            rows32[r, pl.ds(k * 32, 16)] = a
            rows32[r, pl.ds(k * 32 + 16, 16)] = b
        return carry

    lax.fori_loop(lo, hi, row, 0)


def _edge_call(np_, nsb0, nsb1, d):
    rpt = np_ // NS
    HF = CH // 2

    def body(src4, dst4, h, z128, accp,
             sbufa, sbufb, dbufa, dbufb, rows0, rows1, rows32, acc,
             sa, sb_, s0, s1, ss0, ss1):
        c = lax.axis_index("c")
        s = lax.axis_index("s")
        wid = c * NS + s
        pltpu.sync_copy(src4.at[wid, 0], sbufa)
        pltpu.sync_copy(dst4.at[wid, 0], dbufa)
        pltpu.async_copy(src4.at[wid, 1], sbufb, sb_)
        pltpu.async_copy(dst4.at[wid, 1], dbufb, sb_)
        r0 = s * rpt
        pltpu.sync_copy(z128, acc.at[pl.ds(r0, rpt)])
        plsc.subcore_barrier()

        half = jnp.where(c == 0, nsb0 // 2, nsb1 // 2)

        def half_wait(sem):
            pltpu.make_async_copy(
                rows32.at[pl.ds(0, HF)], accp.at[0, pl.ds(0, HF)], sem).wait()

        def process(sbuf, dbuf, prev):
            pltpu.async_copy(h.at[sbuf.at[0]], rows0, s0)
            for k in range(SB):
                rw, sw = (rows0, s0) if k % 2 == 0 else (rows1, s1)
                pltpu.make_async_copy(h.at[sbuf.at[k]], rw, sw).wait()
                if k + 1 < SB:
                    nrw, nsw = (rows1, s1) if k % 2 == 0 else (rows0, s0)
                    pltpu.async_copy(h.at[sbuf.at[k + 1]], nrw, nsw)
                for hh, ssem in ((0, ss0), (1, ss1)):
                    if k > 0 or prev is True:
                        half_wait(ssem)
                    elif prev is not False:
                        @pl.when(prev)
                        def _():
                            half_wait(ssem)
                    _unpack_rows(rw, rows32, d, hh * HF, (hh + 1) * HF)
                    pltpu.async_copy(rows32.at[pl.ds(hh * HF, HF)],
                                     acc.at[dbuf.at[2 * k + hh]], ssem,
                                     add=True)

        def step(g, carry):
            @pl.when(g > 0)
            def _():
                pltpu.make_async_copy(src4.at[wid, 0], sbufa, sa).wait()
                pltpu.make_async_copy(dst4.at[wid, 0], dbufa, sa).wait()

            process(sbufa, dbufa, g > 0)

            @pl.when(g + 1 < half)
            def _():
                pltpu.async_copy(src4.at[wid, 2 * g + 2], sbufa, sa)
                pltpu.async_copy(dst4.at[wid, 2 * g + 2], dbufa, sa)

            pltpu.make_async_copy(src4.at[wid, 1], sbufb, sb_).wait()
            pltpu.make_async_copy(dst4.at[wid, 1], dbufb, sb_).wait()
            process(sbufb, dbufb, True)

            @pl.when(g + 1 < half)
            def _():
                pltpu.async_copy(src4.at[wid, 2 * g + 3], sbufb, sb_)
                pltpu.async_copy(dst4.at[wid, 2 * g + 3], dbufb, sb_)

            return carry

        lax.fori_loop(0, half, step, 0)
        half_wait(ss0)
        half_wait(ss1)
        plsc.subcore_barrier()
        pltpu.sync_copy(acc.at[pl.ds(r0, rpt)], accp.at[c, pl.ds(r0, rpt)])

    return pl.kernel(
        body,
        out_type=jax.ShapeDtypeStruct((NC, np_, d), jnp.float32),
        mesh=_sc_mesh(),
        compiler_params=pltpu.CompilerParams(needs_layout_passes=False,
                                             use_tc_tiling_on_sc=False),
        scratch_types=[
            pltpu.VMEM((SB, CH), jnp.int32),
            pltpu.VMEM((SB, CH), jnp.int32),
            pltpu.VMEM((2 * SB, CH // 2), jnp.int32),
            pltpu.VMEM((2 * SB, CH // 2), jnp.int32),
            pltpu.VMEM((CH, d // 2), jnp.int32),
            pltpu.VMEM((CH, d // 2), jnp.int32),
            pltpu.VMEM((CH, d), jnp.float32),
            pltpu.VMEM_SHARED((np_, d), jnp.float32),
            pltpu.SemaphoreType.DMA,
            pltpu.SemaphoreType.DMA,
            pltpu.SemaphoreType.DMA,
            pltpu.SemaphoreType.DMA,
            pltpu.SemaphoreType.DMA,
            pltpu.SemaphoreType.DMA,
        ],
    )


def _norms(dvec):
    d0 = dvec[:, None]
    return jnp.where(d0 > 0, lax.rsqrt(jnp.maximum(d0, 1.0)), 0.0)


def _tc_pre(np_, d, blk):
    grid = np_ // blk

    def body(degp_ref, x_ref, w_ref, ns_ref, nd_ref, h_ref):
        dp = degp_ref[...]
        ns = _norms(dp[:, :, 0, :].sum((0, 1)))
        nd = _norms(dp[:, :, 1, :].sum((0, 1)))
        ns_ref[...] = ns
        nd_ref[...] = nd
        h_ref[...] = jnp.dot(x_ref[...] * ns, w_ref[...],
                             preferred_element_type=jnp.float32
                             ).astype(jnp.bfloat16)

    return pl.pallas_call(
        body,
        grid=(grid,),
        in_specs=[
            pl.BlockSpec((NC, NS, 2, blk), lambda i: (0, 0, 0, i)),
            pl.BlockSpec((blk, d), lambda i: (i, 0)),
            pl.BlockSpec((d, d), lambda i: (0, 0)),
        ],
        out_specs=[
            pl.BlockSpec((blk, 1), lambda i: (i, 0)),
            pl.BlockSpec((blk, 1), lambda i: (i, 0)),
            pl.BlockSpec((blk, d), lambda i: (i, 0)),
        ],
        out_shape=[
            jax.ShapeDtypeStruct((np_, 1), jnp.float32),
            jax.ShapeDtypeStruct((np_, 1), jnp.float32),
            jax.ShapeDtypeStruct((np_, d), jnp.bfloat16),
        ],
    )


def _tc_mid(np_, d, blk):
    grid = np_ // blk

    def body(accp_ref, ns_ref, nd_ref, b_ref, w_ref, h_ref):
        ap = accp_ref[...]
        z = jnp.maximum((ap[0] + ap[1]) * nd_ref[...] + b_ref[...], 0.0)
        h_ref[...] = jnp.dot(z * ns_ref[...], w_ref[...],
                             preferred_element_type=jnp.float32
                             ).astype(jnp.bfloat16)

    return pl.pallas_call(
        body,
        grid=(grid,),
        in_specs=[
            pl.BlockSpec((NC, blk, d), lambda i: (0, i, 0)),
            pl.BlockSpec((blk, 1), lambda i: (i, 0)),
            pl.BlockSpec((blk, 1), lambda i: (i, 0)),
            pl.BlockSpec((1, d), lambda i: (0, 0)),
            pl.BlockSpec((d, d), lambda i: (0, 0)),
        ],
        out_specs=pl.BlockSpec((blk, d), lambda i: (i, 0)),
        out_shape=jax.ShapeDtypeStruct((np_, d), jnp.bfloat16),
    )


def _tc_post(n, d, blk):
    grid = n // blk

    def body(accp_ref, nd_ref, b_ref, out_ref):
        ap = accp_ref[...]
        out_ref[...] = jnp.maximum((ap[0] + ap[1]) * nd_ref[...] + b_ref[...], 0.0)

    return pl.pallas_call(
        body,
        grid=(grid,),
        in_specs=[
            pl.BlockSpec((NC, blk, d), lambda i: (0, i, 0)),
            pl.BlockSpec((blk, 1), lambda i: (i, 0)),
            pl.BlockSpec((1, d), lambda i: (0, 0)),
        ],
        out_specs=pl.BlockSpec((blk, d), lambda i: (i, 0)),
        out_shape=jax.ShapeDtypeStruct((n, d), jnp.float32),
    )


def kernel(x, edge_index, W1, b1, W2, b2):
    n, d = x.shape
    e = edge_index.shape[1]

    np_ = ((n + 1 + 2047) // 2048) * 2048
    gran = NW * CH * SB * 2
    ep = ((e + gran - 1) // gran) * gran
    tpc = ep // (NW * CH)
    nsb = tpc // SB
    nsb1 = 2 * max(2, int(round(nsb * 2 * 0.4 / 2)))
    nsb0 = 2 * nsb - nsb1
    rpt = np_ // NS

    pad = jnp.full((ep - e,), n, dtype=jnp.int32)
    src3 = jnp.concatenate([edge_index[0], pad]).reshape(NW, tpc, CH)
    dst3 = jnp.concatenate([edge_index[1], pad]).reshape(NW, tpc, CH)
    idx2 = jnp.stack([src3, dst3], axis=2).reshape(NW, 2 * tpc, CH)

    spe = SB * CH
    cut = NS * nsb0 * spe
    srcp = jnp.concatenate([edge_index[0], pad])
    dstp = jnp.concatenate([edge_index[1], pad])

    nsbm = max(nsb0, nsb1)

    def uneven(v):
        a = v[:cut].reshape(NS, nsb0, spe)
        a = jnp.pad(a, ((0, 0), (0, nsbm - nsb0), (0, 0)), constant_values=n)
        b = v[cut:].reshape(NS, nsb1, spe)
        b = jnp.pad(b, ((0, 0), (0, nsbm - nsb1), (0, 0)), constant_values=n)
        return jnp.concatenate([a, b], axis=0)

    src4 = uneven(srcp).reshape(NW, nsbm, SB, CH)
    dst4 = uneven(dstp).reshape(NW, nsbm, 2 * SB, CH // 2)

    z128 = jnp.zeros((rpt, d), dtype=jnp.float32)
    xp = jnp.pad(x, ((0, np_ - n), (0, 0)))
    b1r = b1.reshape(1, d)
    b2r = b2.reshape(1, d)

    perm = []
    for g in range(d // 32):
        perm += [32 * g + 2 * i for i in range(16)]
        perm += [32 * g + 2 * i + 1 for i in range(16)]
    q = np.argsort(np.asarray(perm))
    W1q = W1[:, q]
    W2q = W2[:, q]

    def pack_i32(hb):
        return lax.bitcast_convert_type(hb.reshape(np_, d // 2, 2), jnp.int32)

    degp = _degree_call(np_, tpc)(idx2)
    ns, nd, h1 = _tc_pre(np_, d, 1024)(degp, xp, W1q)
    acc1 = _edge_call(np_, nsb0, nsb1, d)(src4, dst4, pack_i32(h1), z128)
    h2 = _tc_mid(np_, d, 1024)(acc1, ns, nd, b1r, W2q)
    acc2 = _edge_call(np_, nsb0, nsb1, d)(src4, dst4, pack_i32(h2), z128)
    out = _tc_post(n, d, 1000)(acc2, nd, b2r)
    return out
